# baseline (speedup 1.0000x reference)
import functools

import jax
import jax.numpy as jnp
from jax import lax
from jax.experimental import pallas as pl
from jax.experimental.pallas import tpu as pltpu
from jax.experimental.pallas import tpu_sc as plsc

_NC = 2
_NS = 16
_NW = _NC * _NS


def _sc_gather(u, f, a, tables):
    B = u.shape[0]
    assert B % (8 * _NW) == 0
    bpw = B // _NW
    dims = [t.shape[1] for _, t in tables]

    mesh = plsc.VectorSubcoreMesh(core_axis_name="c", subcore_axis_name="s")

    aligned = []
    ragged = list(range(len(dims)))

    @functools.partial(
        pl.kernel,
        mesh=mesh,
        out_type=[jax.ShapeDtypeStruct((B, d), jnp.float32) for d in dims],
        scratch_types=(
            [pltpu.VMEM((bpw,), jnp.int32) for _ in range(3)]
            + [pltpu.VMEM((bpw, d), jnp.float32) for d in dims]
            + [pltpu.SemaphoreType.DMA]
        ),
    )
    def gather_kernel(*args):
        nt = len(dims)
        u_hbm, f_hbm, a_hbm = args[0:3]
        tabs = args[3:3 + nt]
        outs = args[3 + nt:3 + 2 * nt]
        idx_v = args[3 + 2 * nt:6 + 2 * nt]
        bufs = args[6 + 2 * nt:6 + 3 * nt]
        sem = args[6 + 3 * nt]

        wid = lax.axis_index("s") * _NC + lax.axis_index("c")
        base = wid * bpw
        pltpu.sync_copy(u_hbm.at[pl.ds(base, bpw)], idx_v[0])
        pltpu.sync_copy(f_hbm.at[pl.ds(base, bpw)], idx_v[1])
        pltpu.sync_copy(a_hbm.at[pl.ds(base, bpw)], idx_v[2])

        which = [0 if (src is u) else (1 if (src is f) else 2)
                 for src, _ in tables]
        copies = []
        for t in aligned:
            copies.append(
                pltpu.async_copy(tabs[t].at[idx_v[which[t]]], bufs[t], sem))

        need = sorted({which[t] for t in ragged})
        scal = {}
        for k in need:
            vals = []
            for g in range(0, bpw, 16):
                vec = idx_v[k][pl.ds(g, 16)]
                for j in range(16):
                    vals.append(vec[j])
            scal[k] = vals
        for t in ragged:
            vals = scal[which[t]]
            for j in range(bpw):
                copies.append(pltpu.async_copy(
                    tabs[t].at[pl.ds(vals[j], 1)], bufs[t].at[pl.ds(j, 1)],
                    sem))

        for c in copies:
            c.wait()
        for t in range(nt):
            pltpu.sync_copy(bufs[t], outs[t].at[pl.ds(base, bpw)])

    return gather_kernel(u, f, a, *[t for _, t in tables])


def _fused_outer_matmul(E, Fstack, W, bias, G, out_dim):
    B, I = E.shape
    J = Fstack.shape[2]
    assert I % G == 0
    ng = I // G
    blk = G * J
    assert blk % 8 == 0

    E3 = E.T.reshape(ng, G, B)

    bound = I // 2
    g_strad = bound // G
    strad_ii = bound - g_strad * G

    def pieces(E_ref, F, lo, hi):
        return [E_ref[0, ii][:, None] * F for ii in range(lo, hi)]

    def body(E_ref, F_ref, W_ref, b_ref, o_ref):
        g = pl.program_id(0)

        @pl.when(g == 0)
        def _init():
            o_ref[...] = jnp.zeros((B, out_dim), jnp.float32) + b_ref[...]

        def accum(xs):
            x = xs[0] if len(xs) == 1 else jnp.concatenate(xs, axis=1)
            o_ref[...] += jnp.dot(x, W_ref[...],
                                  preferred_element_type=jnp.float32)

        F1 = F_ref[0]
        F2 = F_ref[1]

        @pl.when(g < g_strad)
        def _lo():
            accum(pieces(E_ref, F1, 0, G))

        if strad_ii > 0:
            @pl.when(g == g_strad)
            def _mid():
                accum(pieces(E_ref, F1, 0, strad_ii)
                      + pieces(E_ref, F2, strad_ii, G))

        @pl.when(g > g_strad if strad_ii > 0 else g >= g_strad)
        def _hi():
            accum(pieces(E_ref, F2, 0, G))

        @pl.when(g == ng - 1)
        def _relu():
            o_ref[...] = jnp.maximum(o_ref[...], 0.0)

    return pl.pallas_call(
        body,
        grid=(ng,),
        in_specs=[
            pl.BlockSpec((1, G, B), lambda g: (g, 0, 0)),
            pl.BlockSpec((2, B, J), lambda g: (0, 0, 0)),
            pl.BlockSpec((blk, out_dim), lambda g: (g, 0)),
            pl.BlockSpec((1, out_dim), lambda g: (0, 0)),
        ],
        out_specs=pl.BlockSpec((B, out_dim), lambda g: (0, 0)),
        out_shape=jax.ShapeDtypeStruct((B, out_dim), jnp.float32),
        compiler_params=pltpu.CompilerParams(
            dimension_semantics=("arbitrary",)),
    )(E3, Fstack, W, bias.reshape(1, out_dim))


def kernel(inputs, tfidf_svd_user_feed, tfidf_svd_feed_user, tfidf_svd_user_author, tfidf_svd_author_user, tfidf_svd_feed_emb, tfidf_svd_tag_user, tfidf_svd_hkey_user, tfidf_svd_mkey_user, tfidf_svd_tag_feed, tfidf_svd_hkey_feed, tfidf_svd_mkey_feed, user_feed_d2v, feed_user_d2v, user_author_d2v, author_user_d2v, first_order_shifts, W256, b256, W128, b128):
    inputs = inputs.reshape(-1, 3)
    one_hot_inputs = inputs + first_order_shifts[None, :]
    u = inputs[:, 0]
    f = inputs[:, 1]
    a = inputs[:, 2]

    tables = [
        (u, tfidf_svd_user_feed),
        (u, tfidf_svd_user_author),
        (u, tfidf_svd_tag_user),
        (u, tfidf_svd_hkey_user),
        (u, tfidf_svd_mkey_user),
        (u, user_feed_d2v),
        (u, user_author_d2v),
        (f, tfidf_svd_feed_user),
        (f, tfidf_svd_tag_feed),
        (f, tfidf_svd_hkey_feed),
        (f, tfidf_svd_feed_emb),
        (f, tfidf_svd_mkey_feed),
        (f, feed_user_d2v),
        (a, tfidf_svd_author_user),
        (a, author_user_d2v),
    ]
    g = _sc_gather(u, f, a, tables)
    (user_feed_embed, user_author_embed, user_tag_embed, user_key1_embed,
     user_key2_embed, user_feed_d2vem, user_author_d2vem, feed_user_embed,
     feed_tag_embed, feed_key1_embed, feed_emb_embed, feed_key2_embed,
     feed_user_d2vem, author_user_embed, author_user_d2vem) = g

    B = inputs.shape[0]

    E256 = jnp.concatenate([user_feed_embed, user_author_embed], axis=1)
    F256 = jnp.stack([feed_user_embed, author_user_embed], axis=0)
    user_x_user = _fused_outer_matmul(
        E256, F256, W256, b256, G=20, out_dim=256)

    E128 = jnp.concatenate([user_feed_d2vem, user_author_d2vem], axis=1)
    F128 = jnp.stack([feed_user_d2vem, author_user_d2vem], axis=0)
    user_w_user = _fused_outer_matmul(
        E128, F128, W128, b128, G=16, out_dim=128)

    embeds_fm = jnp.stack([user_feed_embed, user_author_embed,
                           feed_user_embed, author_user_embed], axis=0)
    embed_inputs = embeds_fm.reshape(-1, 4, 150)

    expert_inputs = jnp.concatenate([
        user_feed_embed, user_author_embed, user_tag_embed, user_key1_embed,
        user_key2_embed, user_feed_d2vem, user_author_d2vem, feed_user_embed,
        feed_tag_embed, feed_key1_embed, feed_emb_embed, feed_key2_embed,
        feed_user_d2vem, author_user_embed, author_user_d2vem, user_x_user,
        user_w_user], axis=-1)

    return (expert_inputs, one_hot_inputs, embed_inputs)


# speedup vs baseline: 1.0296x; 1.0296x over previous
import functools

import jax
import jax.numpy as jnp
from jax import lax
from jax.experimental import pallas as pl
from jax.experimental.pallas import tpu as pltpu
from jax.experimental.pallas import tpu_sc as plsc

_NC = 2
_NS = 16
_NW = _NC * _NS


def _sc_gather(u, f, a, tables):
    B = u.shape[0]
    assert B % (8 * _NW) == 0
    bpw = B // _NW
    dims = [t.shape[1] for _, t in tables]

    mesh = plsc.VectorSubcoreMesh(core_axis_name="c", subcore_axis_name="s")

    aligned = []
    ragged = list(range(len(dims)))

    @functools.partial(
        pl.kernel,
        mesh=mesh,
        out_type=[jax.ShapeDtypeStruct((B, d), jnp.float32) for d in dims],
        scratch_types=(
            [pltpu.VMEM((bpw,), jnp.int32) for _ in range(3)]
            + [pltpu.VMEM((bpw, d), jnp.float32) for d in dims]
            + [pltpu.SemaphoreType.DMA]
        ),
    )
    def gather_kernel(*args):
        nt = len(dims)
        u_hbm, f_hbm, a_hbm = args[0:3]
        tabs = args[3:3 + nt]
        outs = args[3 + nt:3 + 2 * nt]
        idx_v = args[3 + 2 * nt:6 + 2 * nt]
        bufs = args[6 + 2 * nt:6 + 3 * nt]
        sem = args[6 + 3 * nt]

        wid = lax.axis_index("s") * _NC + lax.axis_index("c")
        base = wid * bpw
        pltpu.sync_copy(u_hbm.at[pl.ds(base, bpw)], idx_v[0])
        pltpu.sync_copy(f_hbm.at[pl.ds(base, bpw)], idx_v[1])
        pltpu.sync_copy(a_hbm.at[pl.ds(base, bpw)], idx_v[2])

        which = [0 if (src is u) else (1 if (src is f) else 2)
                 for src, _ in tables]
        copies = []
        for t in aligned:
            copies.append(
                pltpu.async_copy(tabs[t].at[idx_v[which[t]]], bufs[t], sem))

        need = sorted({which[t] for t in ragged})
        scal = {}
        for k in need:
            vals = []
            for g in range(0, bpw, 16):
                vec = idx_v[k][pl.ds(g, 16)]
                for j in range(16):
                    vals.append(vec[j])
            scal[k] = vals
        for t in ragged:
            vals = scal[which[t]]
            for j in range(bpw):
                copies.append(pltpu.async_copy(
                    tabs[t].at[pl.ds(vals[j], 1)], bufs[t].at[pl.ds(j, 1)],
                    sem))

        for c in copies:
            c.wait()
        for t in range(nt):
            pltpu.sync_copy(bufs[t], outs[t].at[pl.ds(base, bpw)])

    return gather_kernel(u, f, a, *[t for _, t in tables])


def _fused_outer_matmul(E, Fstack, W, bias, G, out_dim):
    B, I = E.shape
    J = Fstack.shape[2]
    assert (I // 2) % G == 0
    ng = I // G
    half = ng // 2
    Wr = W.reshape(I, J, out_dim)

    E3 = E.T.reshape(ng, G, B)

    def body(E_ref, F_ref, W_ref, b_ref, o_ref):
        g = pl.program_id(0)

        @pl.when(g == 0)
        def _init():
            o_ref[...] = jnp.zeros((B, out_dim), jnp.float32) + b_ref[...]

        F = F_ref[0]
        acc = None
        for ii in range(G):
            x = E_ref[0, ii][:, None] * F
            p = jnp.dot(x, W_ref[ii], preferred_element_type=jnp.float32)
            acc = p if acc is None else acc + p
        o_ref[...] += acc

        @pl.when(g == ng - 1)
        def _relu():
            o_ref[...] = jnp.maximum(o_ref[...], 0.0)

    return pl.pallas_call(
        body,
        grid=(ng,),
        in_specs=[
            pl.BlockSpec((1, G, B), lambda g: (g, 0, 0)),
            pl.BlockSpec((1, B, J), lambda g: (g // half, 0, 0)),
            pl.BlockSpec((G, J, out_dim), lambda g: (g, 0, 0)),
            pl.BlockSpec((1, out_dim), lambda g: (0, 0)),
        ],
        out_specs=pl.BlockSpec((B, out_dim), lambda g: (0, 0)),
        out_shape=jax.ShapeDtypeStruct((B, out_dim), jnp.float32),
        compiler_params=pltpu.CompilerParams(
            dimension_semantics=("arbitrary",)),
    )(E3, Fstack, Wr, bias.reshape(1, out_dim))


def kernel(inputs, tfidf_svd_user_feed, tfidf_svd_feed_user, tfidf_svd_user_author, tfidf_svd_author_user, tfidf_svd_feed_emb, tfidf_svd_tag_user, tfidf_svd_hkey_user, tfidf_svd_mkey_user, tfidf_svd_tag_feed, tfidf_svd_hkey_feed, tfidf_svd_mkey_feed, user_feed_d2v, feed_user_d2v, user_author_d2v, author_user_d2v, first_order_shifts, W256, b256, W128, b128):
    inputs = inputs.reshape(-1, 3)
    one_hot_inputs = inputs + first_order_shifts[None, :]
    u = inputs[:, 0]
    f = inputs[:, 1]
    a = inputs[:, 2]

    tables = [
        (u, tfidf_svd_user_feed),
        (u, tfidf_svd_user_author),
        (u, tfidf_svd_tag_user),
        (u, tfidf_svd_hkey_user),
        (u, tfidf_svd_mkey_user),
        (u, user_feed_d2v),
        (u, user_author_d2v),
        (f, tfidf_svd_feed_user),
        (f, tfidf_svd_tag_feed),
        (f, tfidf_svd_hkey_feed),
        (f, tfidf_svd_feed_emb),
        (f, tfidf_svd_mkey_feed),
        (f, feed_user_d2v),
        (a, tfidf_svd_author_user),
        (a, author_user_d2v),
    ]
    g = _sc_gather(u, f, a, tables)
    (user_feed_embed, user_author_embed, user_tag_embed, user_key1_embed,
     user_key2_embed, user_feed_d2vem, user_author_d2vem, feed_user_embed,
     feed_tag_embed, feed_key1_embed, feed_emb_embed, feed_key2_embed,
     feed_user_d2vem, author_user_embed, author_user_d2vem) = g

    B = inputs.shape[0]

    E256 = jnp.concatenate([user_feed_embed, user_author_embed], axis=1)
    F256 = jnp.stack([feed_user_embed, author_user_embed], axis=0)
    user_x_user = _fused_outer_matmul(
        E256, F256, W256, b256, G=10, out_dim=256)

    E128 = jnp.concatenate([user_feed_d2vem, user_author_d2vem], axis=1)
    F128 = jnp.stack([feed_user_d2vem, author_user_d2vem], axis=0)
    user_w_user = _fused_outer_matmul(
        E128, F128, W128, b128, G=16, out_dim=128)

    embeds_fm = jnp.stack([user_feed_embed, user_author_embed,
                           feed_user_embed, author_user_embed], axis=0)
    embed_inputs = embeds_fm.reshape(-1, 4, 150)

    expert_inputs = jnp.concatenate([
        user_feed_embed, user_author_embed, user_tag_embed, user_key1_embed,
        user_key2_embed, user_feed_d2vem, user_author_d2vem, feed_user_embed,
        feed_tag_embed, feed_key1_embed, feed_emb_embed, feed_key2_embed,
        feed_user_d2vem, author_user_embed, author_user_d2vem, user_x_user,
        user_w_user], axis=-1)

    return (expert_inputs, one_hot_inputs, embed_inputs)
